# baseline (speedup 1.0000x reference)
import functools

import jax
import jax.numpy as jnp
from jax import lax
from jax.experimental import pallas as pl
from jax.experimental.pallas import tpu as pltpu
from jax.experimental.pallas import tpu_sc as plsc

NC = 2
NS = 16
NW = NC * NS
LANES = 16
CHUNK = 256
S_SC = 8192
R_TC = 512

NEG = float("-inf")


def _sc_stage(flat1d, cu_lo, cu_hi, d, b):
    rows_w = S_SC // NW
    nchunk = max(rows_w // CHUNK, 1)
    chunk = min(CHUNK, rows_w)
    mesh = plsc.VectorSubcoreMesh(
        core_axis_name="c", subcore_axis_name="s", num_cores=NC, num_subcores=NS
    )

    @functools.partial(
        pl.kernel,
        out_type=jax.ShapeDtypeStruct((NW * b * d,), jnp.float32),
        mesh=mesh,
        scratch_types=[
            pltpu.VMEM((chunk * d,), jnp.float32),
            pltpu.VMEM((chunk * d,), jnp.float32),
            pltpu.VMEM((b,), jnp.int32),
            pltpu.VMEM((b,), jnp.int32),
            pltpu.VMEM((b * d,), jnp.float32),
            pltpu.SemaphoreType.DMA,
            pltpu.SemaphoreType.DMA,
        ],
    )
    def k(flat_hbm, st_hbm, en_hbm, out_hbm, buf0, buf1, st_v, en_v, acc_v, sem0, sem1):
        cid = lax.axis_index("c")
        sid = lax.axis_index("s")
        wid = sid * NC + cid
        base = wid * rows_w * d

        bufs = (buf0, buf1)
        sems = (sem0, sem1)
        pltpu.make_async_copy(flat_hbm.at[pl.ds(base, chunk * d)], buf0, sem0).start()

        pltpu.sync_copy(st_hbm.at[pl.ds(0, b)], st_v)
        pltpu.sync_copy(en_hbm.at[pl.ds(0, b)], en_v)
        wlo = wid * rows_w
        st_vec = jnp.clip(st_v[...] - wlo, 0, rows_w)
        en_vec = jnp.clip(en_v[...] - wlo, 0, rows_w)

        neg = jnp.full((LANES,), NEG, jnp.float32)
        for kk in range(b * d // LANES):
            acc_v[pl.ds(kk * LANES, LANES)] = neg

        for c in range(nchunk):
            buf = bufs[c % 2]
            sem = sems[c % 2]
            pltpu.make_async_copy(
                flat_hbm.at[pl.ds(base + c * chunk * d, chunk * d)], buf, sem
            ).wait()
            if c + 1 < nchunk:
                pltpu.make_async_copy(
                    flat_hbm.at[pl.ds(base + (c + 1) * chunk * d, chunk * d)],
                    bufs[(c + 1) % 2],
                    sems[(c + 1) % 2],
                ).start()
            for s in range(b):
                lo = jnp.maximum(st_vec[s] - c * chunk, 0)
                hi = jnp.minimum(en_vec[s] - c * chunk, chunk)
                accs = tuple(
                    acc_v[pl.ds(s * d + LANES * j, LANES)] for j in range(d // LANES)
                )

                def rbody(r, a, buf=buf):
                    off = r * d
                    return tuple(
                        jnp.maximum(aj, buf[pl.ds(off + LANES * j, LANES)])
                        for j, aj in enumerate(a)
                    )

                accs = plsc.parallel_loop(lo, hi, unroll=4, carry=accs)(rbody)
                for j in range(d // LANES):
                    acc_v[pl.ds(s * d + LANES * j, LANES)] = accs[j]

        pltpu.sync_copy(acc_v, out_hbm.at[pl.ds(wid * b * d, b * d)])

    return k(flat1d, cu_lo, cu_hi)


def _tc_rows(flat, cu_seqlens, n, d, b):
    nblk = (n - S_SC) // R_TC
    blk0 = S_SC // R_TC

    def body(cu_ref, x_ref, o_ref):
        i = pl.program_id(0)

        @pl.when(i == 0)
        def _():
            o_ref[...] = jnp.full((b, d), NEG, jnp.float32)

        r0 = S_SC + i * R_TC
        x = x_ref[...]
        s0 = jnp.int32(0)
        s1 = jnp.int32(0)
        for t in range(1, b):
            ct = cu_ref[t]
            s0 = s0 + jnp.where(ct <= r0, 1, 0).astype(jnp.int32)
            s1 = s1 + jnp.where(ct <= r0 + R_TC - 1, 1, 0).astype(jnp.int32)
        rowid = r0 + lax.broadcasted_iota(jnp.int32, (R_TC, 1), 0)
        segid = lax.broadcasted_iota(jnp.int32, (b, 1), 0)

        def sbody(s, _):
            lo = cu_ref[s]
            hi = cu_ref[s + 1]
            m = (rowid >= lo) & (rowid < hi)
            seg_max = jnp.max(
                jnp.where(m, x, NEG), axis=0, keepdims=True
            )
            upd = jnp.maximum(o_ref[...], seg_max)
            o_ref[...] = jnp.where(segid == s, upd, o_ref[...])
            return 0

        lax.fori_loop(s0, s1 + 1, sbody, 0)

    return pl.pallas_call(
        body,
        grid=(nblk,),
        in_specs=[
            pl.BlockSpec(memory_space=pltpu.SMEM),
            pl.BlockSpec((R_TC, d), lambda i: (blk0 + i, 0)),
        ],
        out_specs=pl.BlockSpec((b, d), lambda i: (0, 0)),
        out_shape=jax.ShapeDtypeStruct((b, d), jnp.float32),
    )(cu_seqlens, flat)


def _tc_merge(partials_sc, partial_tc, b, d):
    def body(p_ref, q_ref, o_ref):
        acc = q_ref[...]
        for w in range(NW):
            acc = jnp.maximum(acc, p_ref[w * b : (w + 1) * b, :])
        o_ref[...] = acc

    return pl.pallas_call(
        body,
        out_shape=jax.ShapeDtypeStruct((b, d), jnp.float32),
    )(partials_sc, partial_tc)


def kernel(flat, cu_seqlens):
    n, d = flat.shape
    b = cu_seqlens.shape[0] - 1
    assert S_SC % NW == 0 and (n - S_SC) % R_TC == 0 and d % LANES == 0

    cu = cu_seqlens.astype(jnp.int32)
    partials_sc = _sc_stage(flat.reshape(-1), cu[:-1], cu[1:], d, b)
    partial_tc = _tc_rows(flat, cu, n, d, b)
    return _tc_merge(partials_sc.reshape(NW * b, d), partial_tc, b, d)
